# Initial kernel scaffold; baseline (speedup 1.0000x reference)
import jax
import jax.numpy as jnp
from jax.experimental import pallas as pl


def kernel(atoms, bonds, edges, W, b):
    raise NotImplementedError("write your pallas kernel here")


# speedup vs baseline: 3.0176x; 3.0176x over previous
import functools

import jax
import jax.numpy as jnp
from jax.experimental import pallas as pl

B, MAX_ATOMS, MAX_DEGREE = 512, 100, 6
NUM_ATOM_FEATURES, NUM_BOND_FEATURES, CONV_WIDTH = 128, 16, 128
BM = 8


def _body(atoms_ref, bonds_ref, edges_ref, W_ref, b_ref, out_ref):
    e = edges_ref[...]
    valid = e >= 0
    deg = jnp.sum(valid.astype(jnp.int32), axis=1, keepdims=True)

    at = atoms_ref[...]

    lane = jax.lax.broadcasted_iota(jnp.int32, (MAX_ATOMS, MAX_ATOMS), 1)
    s_atom_parts = []
    for m in range(BM):
        sl = slice(m * MAX_ATOMS, (m + 1) * MAX_ATOMS)
        e_m = e[sl, :]
        at_m = at[sl, :]
        amat = jnp.zeros((MAX_ATOMS, MAX_ATOMS), dtype=jnp.float32)
        for d in range(MAX_DEGREE):
            amat = amat + jnp.where(e_m[:, d:d + 1] == lane, 1.0, 0.0)
        s_atom_parts.append(
            at_m
            + jax.lax.dot(amat, at_m, precision=jax.lax.Precision.HIGHEST,
                          preferred_element_type=jnp.float32)
        )
    s_atom = jnp.concatenate(s_atom_parts, axis=0)

    bsel_i = jax.lax.broadcasted_iota(jnp.int32, (MAX_DEGREE * NUM_BOND_FEATURES, NUM_BOND_FEATURES), 0)
    bsel_j = jax.lax.broadcasted_iota(jnp.int32, (MAX_DEGREE * NUM_BOND_FEATURES, NUM_BOND_FEATURES), 1)
    bsel = jnp.where(bsel_i % NUM_BOND_FEATURES == bsel_j, 1.0, 0.0)
    s_bond = jax.lax.dot(bonds_ref[...], bsel, precision=jax.lax.Precision.HIGHEST,
                         preferred_element_type=jnp.float32)

    x = jnp.concatenate([s_atom, s_bond], axis=1)

    acc = jnp.zeros((BM * MAX_ATOMS, CONV_WIDTH), dtype=jnp.float32)
    for d in range(MAX_DEGREE):
        y = jax.lax.dot(x, W_ref[d], precision=jax.lax.Precision.HIGHEST,
                        preferred_element_type=jnp.float32)
        y = jax.nn.relu(y + b_ref[d][None, :])
        acc = acc + jnp.where(deg == d, y, 0.0)
    out_ref[...] = acc


@jax.jit
def kernel(atoms, bonds, edges, W, b):
    atoms2d = atoms.reshape(B * MAX_ATOMS, NUM_ATOM_FEATURES)
    bonds2d = bonds.reshape(B * MAX_ATOMS, MAX_DEGREE * NUM_BOND_FEATURES)
    edges2d = edges.reshape(B * MAX_ATOMS, MAX_DEGREE).astype(jnp.int32)

    out = pl.pallas_call(
        _body,
        grid=(B // BM,),
        in_specs=[
            pl.BlockSpec((BM * MAX_ATOMS, NUM_ATOM_FEATURES), lambda i: (i, 0)),
            pl.BlockSpec((BM * MAX_ATOMS, MAX_DEGREE * NUM_BOND_FEATURES), lambda i: (i, 0)),
            pl.BlockSpec((BM * MAX_ATOMS, MAX_DEGREE), lambda i: (i, 0)),
            pl.BlockSpec((MAX_DEGREE, NUM_ATOM_FEATURES + NUM_BOND_FEATURES, CONV_WIDTH),
                         lambda i: (0, 0, 0)),
            pl.BlockSpec((MAX_DEGREE, CONV_WIDTH), lambda i: (0, 0)),
        ],
        out_specs=pl.BlockSpec((BM * MAX_ATOMS, CONV_WIDTH), lambda i: (i, 0)),
        out_shape=jax.ShapeDtypeStruct((B * MAX_ATOMS, CONV_WIDTH), jnp.float32),
    )(atoms2d, bonds2d, edges2d, W, b)
    return out.reshape(B, MAX_ATOMS, CONV_WIDTH)


# speedup vs baseline: 5.2548x; 1.7414x over previous
import functools

import jax
import jax.numpy as jnp
from jax.experimental import pallas as pl

B, MAX_ATOMS, MAX_DEGREE = 512, 100, 6
NUM_ATOM_FEATURES, NUM_BOND_FEATURES, CONV_WIDTH = 128, 16, 128
BM = 8


def _body(atoms_ref, bonds_ref, edges_ref, W_ref, b_ref, out_ref):
    e = edges_ref[...]
    valid = e >= 0
    deg = jnp.sum(valid.astype(jnp.int32), axis=1, keepdims=True)

    at = atoms_ref[...]

    lane = jax.lax.broadcasted_iota(jnp.int32, (MAX_ATOMS, MAX_ATOMS), 1)
    s_atom_parts = []
    for m in range(BM):
        sl = slice(m * MAX_ATOMS, (m + 1) * MAX_ATOMS)
        e_m = e[sl, :]
        at_m = at[sl, :]
        amat = jnp.zeros((MAX_ATOMS, MAX_ATOMS), dtype=jnp.float32)
        for d in range(MAX_DEGREE):
            amat = amat + jnp.where(e_m[:, d:d + 1] == lane, 1.0, 0.0)
        s_atom_parts.append(
            at_m
            + jax.lax.dot(amat, at_m, precision=jax.lax.Precision.DEFAULT,
                          preferred_element_type=jnp.float32)
        )
    s_atom = jnp.concatenate(s_atom_parts, axis=0)

    bsel_i = jax.lax.broadcasted_iota(jnp.int32, (MAX_DEGREE * NUM_BOND_FEATURES, NUM_BOND_FEATURES), 0)
    bsel_j = jax.lax.broadcasted_iota(jnp.int32, (MAX_DEGREE * NUM_BOND_FEATURES, NUM_BOND_FEATURES), 1)
    bsel = jnp.where(bsel_i % NUM_BOND_FEATURES == bsel_j, 1.0, 0.0)
    s_bond = jax.lax.dot(bonds_ref[...], bsel, precision=jax.lax.Precision.DEFAULT,
                         preferred_element_type=jnp.float32)

    x = jnp.concatenate([s_atom, s_bond], axis=1)

    acc = jnp.zeros((BM * MAX_ATOMS, CONV_WIDTH), dtype=jnp.float32)
    for d in range(MAX_DEGREE):
        y = jax.lax.dot(x, W_ref[d], precision=jax.lax.Precision.DEFAULT,
                        preferred_element_type=jnp.float32)
        y = jax.nn.relu(y + b_ref[d][None, :])
        acc = acc + jnp.where(deg == d, y, 0.0)
    out_ref[...] = acc


@jax.jit
def kernel(atoms, bonds, edges, W, b):
    atoms2d = atoms.reshape(B * MAX_ATOMS, NUM_ATOM_FEATURES)
    bonds2d = bonds.reshape(B * MAX_ATOMS, MAX_DEGREE * NUM_BOND_FEATURES)
    edges2d = edges.reshape(B * MAX_ATOMS, MAX_DEGREE).astype(jnp.int32)

    out = pl.pallas_call(
        _body,
        grid=(B // BM,),
        in_specs=[
            pl.BlockSpec((BM * MAX_ATOMS, NUM_ATOM_FEATURES), lambda i: (i, 0)),
            pl.BlockSpec((BM * MAX_ATOMS, MAX_DEGREE * NUM_BOND_FEATURES), lambda i: (i, 0)),
            pl.BlockSpec((BM * MAX_ATOMS, MAX_DEGREE), lambda i: (i, 0)),
            pl.BlockSpec((MAX_DEGREE, NUM_ATOM_FEATURES + NUM_BOND_FEATURES, CONV_WIDTH),
                         lambda i: (0, 0, 0)),
            pl.BlockSpec((MAX_DEGREE, CONV_WIDTH), lambda i: (0, 0)),
        ],
        out_specs=pl.BlockSpec((BM * MAX_ATOMS, CONV_WIDTH), lambda i: (i, 0)),
        out_shape=jax.ShapeDtypeStruct((B * MAX_ATOMS, CONV_WIDTH), jnp.float32),
    )(atoms2d, bonds2d, edges2d, W, b)
    return out.reshape(B, MAX_ATOMS, CONV_WIDTH)
